# baseline (speedup 1.0000x reference)
import jax
import jax.numpy as jnp
from jax import lax
from jax.experimental import pallas as pl
from jax.experimental.pallas import tpu as pltpu
from jax.experimental.pallas import tpu_sc as plsc

N = 10000
E = 320000
NC = 2
NS = 16
NW = NC * NS
L = 16
NEG = 0.2

RANGE = 320
PADR = 336
SENT = RANGE
OUT_P = NW * RANGE
ADP = OUT_P

SLABP = 2000
BUFP = 4224
HALF = 167936
ECAP = 2 * HALF
LSLAB = 2048

_mesh = plsc.VectorSubcoreMesh(core_axis_name="c", subcore_axis_name="s")
_params = pltpu.CompilerParams(needs_layout_passes=False)


def _leaky(v):
    return jnp.where(v >= 0, v, NEG * v)


def _wid():
    return lax.axis_index("s") * NC + lax.axis_index("c")


def _pbody(esrc_hbm, edst_hbm, src_out, dloc_out, cnt_out,
           se0, de0, se1, de1, src_a, dloc_a, src_c, dloc_c, cv, sem0, sem1):
    wid = _wid()
    base = wid * RANGE
    io = lax.iota(jnp.int32, L)

    def issue(q, se, de, sem):
        off = pl.multiple_of(q * SLABP, 8)
        pltpu.async_copy(esrc_hbm.at[pl.ds(off, SLABP)], se, sem)
        pltpu.async_copy(edst_hbm.at[pl.ds(off, SLABP)], de, sem)

    def wait(se, de, sem):
        pltpu.make_async_copy(esrc_hbm.at[pl.ds(0, SLABP)], se, sem).wait()
        pltpu.make_async_copy(edst_hbm.at[pl.ds(0, SLABP)], de, sem).wait()

    def keep(se, de, i, sb, db, cnt):
        s_v = se[pl.ds(i * L, L)]
        d_v = de[pl.ds(i * L, L)]
        dl = d_v - base
        m = (d_v >= base) & (d_v < base + RANGE)
        plsc.store_compressed(sb.at[pl.ds(cnt, L)], s_v, mask=m)
        plsc.store_compressed(db.at[pl.ds(cnt, L)], dl, mask=m)
        return cnt + plsc.all_reduce_population_count(m)[0]

    def scan(se, de, carry):
        def vec(i, carry):
            ca, cb = carry
            ca = keep(se, de, 2 * i, src_a, dloc_a, ca)
            cb = keep(se, de, 2 * i + 1, src_c, dloc_c, cb)
            return (ca, cb)
        ca, cb = lax.fori_loop(0, SLABP // L // 2, vec, (carry[0], carry[2]))
        ca = keep(se, de, SLABP // L - 1, src_a, dloc_a, ca)
        return (ca, carry[1], cb, carry[3])

    def flush1(sb, db, cnt, flushed, hoff):
        full = cnt >= 2048

        @pl.when(full)
        def _():
            off = pl.multiple_of(hoff + flushed, 2048)
            pltpu.sync_copy(sb.at[pl.ds(0, 2048)],
                            src_out.at[pl.ds(off, 2048)])
            pltpu.sync_copy(db.at[pl.ds(0, 2048)],
                            dloc_out.at[pl.ds(off, 2048)])
        nmv = jnp.where(full, (cnt - 2048 + (L - 1)) // L, 0)

        def mv(i, _):
            sb[pl.ds(i * L, L)] = sb[pl.ds(2048 + i * L, L)]
            db[pl.ds(i * L, L)] = db[pl.ds(2048 + i * L, L)]
            return 0
        lax.fori_loop(0, nmv, mv, 0)
        return (jnp.where(full, cnt - 2048, cnt),
                jnp.where(full, flushed + 2048, flushed))

    def flush(carry):
        ca, fa, cb, fb = carry
        ca, fa = flush1(src_a, dloc_a, ca, fa, wid * ECAP)
        cb, fb = flush1(src_c, dloc_c, cb, fb, wid * ECAP + HALF)
        return (ca, fa, cb, fb)

    NPAIR = (E // SLABP) // 2
    issue(0, se0, de0, sem0)

    def pair(p, carry):
        wait(se0, de0, sem0)
        issue(2 * p + 1, se1, de1, sem1)
        carry = flush(scan(se0, de0, carry))
        wait(se1, de1, sem1)

        @pl.when(p + 1 < NPAIR)
        def _():
            issue(2 * p + 2, se0, de0, sem0)
        return flush(scan(se1, de1, carry))
    z = jnp.int32(0)
    ca, fa, cb, fb = lax.fori_loop(0, NPAIR, pair, (z, z, z, z))

    zi = jnp.zeros((L,), jnp.int32)
    sent = jnp.full((L,), SENT, jnp.int32)
    for t in range(8):
        src_a[pl.ds(ca + t * L, L)] = zi
        dloc_a[pl.ds(ca + t * L, L)] = sent
        src_c[pl.ds(cb + t * L, L)] = zi
        dloc_c[pl.ds(cb + t * L, L)] = sent
    ca128 = ((ca + 127) // 128) * 128
    cb128 = ((cb + 127) // 128) * 128
    offa = pl.multiple_of(wid * ECAP + fa, 2048)
    pltpu.sync_copy(src_a, src_out.at[pl.ds(offa, BUFP)])
    pltpu.sync_copy(dloc_a, dloc_out.at[pl.ds(offa, BUFP)])
    offb = pl.multiple_of(wid * ECAP + HALF + fb, 2048)
    pltpu.sync_copy(src_c, src_out.at[pl.ds(offb, BUFP)])
    pltpu.sync_copy(dloc_c, dloc_out.at[pl.ds(offb, BUFP)])
    cva = jnp.broadcast_to(fa + ca128, (L,))
    cvb = jnp.broadcast_to(fb + cb128, (L,))
    cv[pl.ds(0, L)] = jnp.where(io == 0, cva, jnp.where(io == 1, cvb, 0))
    pltpu.sync_copy(cv, cnt_out.at[pl.ds(pl.multiple_of(wid * L, L), L)])


_sc_partition = pl.kernel(
    _pbody,
    out_type=(
        jax.ShapeDtypeStruct((NW * ECAP,), jnp.int32),
        jax.ShapeDtypeStruct((NW * ECAP,), jnp.int32),
        jax.ShapeDtypeStruct((NW * L,), jnp.int32),
    ),
    mesh=_mesh,
    compiler_params=_params,
    scratch_types=[
        pltpu.VMEM((SLABP,), jnp.int32),
        pltpu.VMEM((SLABP,), jnp.int32),
        pltpu.VMEM((SLABP,), jnp.int32),
        pltpu.VMEM((SLABP,), jnp.int32),
        pltpu.VMEM((BUFP,), jnp.int32),
        pltpu.VMEM((BUFP,), jnp.int32),
        pltpu.VMEM((BUFP,), jnp.int32),
        pltpu.VMEM((BUFP,), jnp.int32),
        pltpu.VMEM((L,), jnp.int32),
        pltpu.SemaphoreType.DMA,
        pltpu.SemaphoreType.DMA,
    ],
)


def _make_sc_edge(D, KCH):
    KV = D // L

    def body(h_hbm, slh, dlh, cth, as_hbm, adp_hbm, b_hbm, out_hbm,
             acc, den, inv320, a_s_v, a_d_own, bias_v,
             src_l, dloc_l, w_l, rows0, rows1, si0, di0, wc0, si1, di1, wc1,
             counts_v, stage16, sem0, sem1):
        wid = _wid()
        base = wid * RANGE
        io = lax.iota(jnp.int32, L)
        zf = jnp.zeros((L,), jnp.float32)

        pltpu.sync_copy(as_hbm, a_s_v)
        pltpu.sync_copy(adp_hbm.at[pl.ds(pl.multiple_of(base, RANGE), RANGE)],
                        a_d_own.at[pl.ds(0, RANGE)])
        pltpu.sync_copy(b_hbm, bias_v)
        pltpu.sync_copy(cth, counts_v)
        cvv = counts_v[pl.ds(wid * L, L)]
        cntA = cvv[0]
        cntB = cvv[1]

        def zrow(j, _):
            for k in range(KV):
                acc[j, pl.ds(k * L, L)] = zf
            return 0
        lax.fori_loop(0, PADR, zrow, 0)

        def zden(t, _):
            den[pl.ds(t * L, L)] = zf
            return 0
        lax.fori_loop(0, 352 // L, zden, 0)

        def mx(i, mv):
            return jnp.maximum(mv, a_s_v[pl.ds(i * L, L)])
        mv = lax.fori_loop(0, N // L, mx, jnp.full((L,), -3.0e38, jnp.float32))
        stage16[pl.ds(0, L)] = mv
        for sh in (8, 4, 2, 1):
            mv = jnp.maximum(mv, plsc.load_gather(stage16, [io ^ sh]))
            stage16[pl.ds(0, L)] = mv
        Msp = mv

        def stage_issue(c, si, di, wc, rows, sem):
            for t in range(KCH // L):
                si[pl.ds(t * L, L)] = src_l[pl.ds(c * KCH + t * L, L)]
                di[pl.ds(t * L, L)] = dloc_l[pl.ds(c * KCH + t * L, L)]
                wc[pl.ds(t * L, L)] = w_l[pl.ds(c * KCH + t * L, L)]
            pltpu.async_copy(h_hbm.at[si], rows, sem)

        def process(di, wc, rows):
            def grp(g, _):
                dl_v = di[pl.ds(g * L, L)]
                w_v = wc[pl.ds(g * L, L)]
                d_all = [dl_v[j] for j in range(L)]
                wsp_all = [jnp.broadcast_to(w_v[j], (L,)) for j in range(L)]
                for j in range(L):
                    d_j = d_all[j]
                    wsp = wsp_all[j]
                    r = g * L + j
                    vals = [rows[r, pl.ds(k * L, L)] * wsp for k in range(KV)]
                    for k in range(KV):
                        plsc.addupdate(acc.at[d_j, pl.ds(k * L, L)], vals[k])
                    dbase = (d_j // L) * L
                    unit = jnp.where(io == (d_j % L), wsp, 0.0)
                    plsc.addupdate(den.at[pl.ds(dbase, L)], unit)
                return 0
            lax.fori_loop(0, KCH // L, grp, 0)

        def do_chunks(nproc):
            def passa(i, _):
                sg = src_l[pl.ds(i * L, L)]
                dl = dloc_l[pl.ds(i * L, L)]
                asg = plsc.load_gather(a_s_v, [sg])
                adg = plsc.load_gather(a_d_own, [dl])
                e = _leaky(asg + adg)
                mt = _leaky(Msp + adg)
                w_l[pl.ds(i * L, L)] = jnp.exp(e - mt)
                return 0
            lax.fori_loop(0, nproc // L, passa, 0)

            npair = nproc // (2 * KCH)
            stage_issue(0, si0, di0, wc0, rows0, sem0)

            def pairb(p, _):
                pltpu.make_async_copy(h_hbm.at[si0], rows0, sem0).wait()
                stage_issue(2 * p + 1, si1, di1, wc1, rows1, sem1)
                process(di0, wc0, rows0)
                pltpu.make_async_copy(h_hbm.at[si1], rows1, sem1).wait()

                @pl.when(p + 1 < npair)
                def _():
                    stage_issue(2 * p + 2, si0, di0, wc0, rows0, sem0)
                process(di1, wc1, rows1)
                return 0
            lax.fori_loop(0, npair, pairb, 0)

        def selfapp(t, cnt):
            g = base + t * L + io
            m = g < N
            plsc.store_compressed(src_l.at[pl.ds(cnt, L)], g, mask=m)
            plsc.store_compressed(dloc_l.at[pl.ds(cnt, L)], t * L + io, mask=m)
            return cnt + plsc.all_reduce_population_count(m)[0]
        cnt = lax.fori_loop(0, RANGE // L, selfapp, jnp.int32(0))
        zi = jnp.zeros((L,), jnp.int32)
        sent = jnp.full((L,), SENT, jnp.int32)
        for t in range(8):
            src_l[pl.ds(cnt + t * L, L)] = zi
            dloc_l[pl.ds(cnt + t * L, L)] = sent
        do_chunks(((cnt + 127) // 128) * 128)

        for hbase, cnt128 in ((wid * ECAP, cntA), (wid * ECAP + HALF, cntB)):
            nslab = (cnt128 + (LSLAB - 1)) // LSLAB

            def slab(q, _, hbase=hbase, cnt128=cnt128):
                off = pl.multiple_of(hbase + q * LSLAB, LSLAB)
                pltpu.sync_copy(slh.at[pl.ds(off, LSLAB)],
                                src_l.at[pl.ds(0, LSLAB)])
                pltpu.sync_copy(dlh.at[pl.ds(off, LSLAB)],
                                dloc_l.at[pl.ds(0, LSLAB)])
                do_chunks(jnp.minimum(LSLAB, cnt128 - q * LSLAB))
                return 0
            lax.fori_loop(0, nslab, slab, 0)

        for t in range(RANGE // L):
            dv = den[pl.ds(t * L, L)]
            inv320[pl.ds(t * L, L)] = 1.0 / (dv + 1e-16)

        def fin(j, _):
            isp = plsc.load_gather(inv320, [jnp.broadcast_to(j, (L,))])
            vals = [acc[j, pl.ds(k * L, L)] * isp + bias_v[pl.ds(k * L, L)]
                    for k in range(KV)]
            for k in range(KV):
                acc[j, pl.ds(k * L, L)] = vals[k]
            return 0
        lax.fori_loop(0, RANGE, fin, 0)
        for g in range(RANGE // 64):
            pltpu.sync_copy(acc.at[pl.ds(g * 64, 64)],
                            out_hbm.at[pl.ds(pl.multiple_of(base + g * 64, 64), 64)])

    return pl.kernel(
        body,
        out_type=jax.ShapeDtypeStruct((OUT_P, D), jnp.float32),
        mesh=_mesh,
        compiler_params=_params,
        scratch_types=[
            pltpu.VMEM((PADR, D), jnp.float32),
            pltpu.VMEM((352,), jnp.float32),
            pltpu.VMEM((RANGE,), jnp.float32),
            pltpu.VMEM((N,), jnp.float32),
            pltpu.VMEM((PADR,), jnp.float32),
            pltpu.VMEM((D,), jnp.float32),
            pltpu.VMEM((LSLAB + 128,), jnp.int32),
            pltpu.VMEM((LSLAB + 128,), jnp.int32),
            pltpu.VMEM((LSLAB + 128,), jnp.float32),
            pltpu.VMEM((KCH, D), jnp.float32),
            pltpu.VMEM((KCH, D), jnp.float32),
            pltpu.VMEM((KCH,), jnp.int32),
            pltpu.VMEM((KCH,), jnp.int32),
            pltpu.VMEM((KCH,), jnp.float32),
            pltpu.VMEM((KCH,), jnp.int32),
            pltpu.VMEM((KCH,), jnp.int32),
            pltpu.VMEM((KCH,), jnp.float32),
            pltpu.VMEM((NW * L,), jnp.int32),
            pltpu.VMEM((L,), jnp.float32),
            pltpu.SemaphoreType.DMA,
            pltpu.SemaphoreType.DMA,
        ],
    )


_sc_edge_256 = _make_sc_edge(256, 32)
_sc_edge_128 = _make_sc_edge(128, 64)


def _make_tc_dense(K, Dout, relu_in):
    BLK = 400
    GRID = N // BLK

    def body(x_ref, w_ref, as_ref, ad_ref, h_ref, asv_ref, adv_ref):
        xb = x_ref[...]
        if relu_in:
            xb = jnp.maximum(xb, 0.0)
        h = jnp.dot(xb, w_ref[...], preferred_element_type=jnp.float32)
        h_ref[...] = h
        asv_ref[...] = jnp.dot(h, as_ref[...], preferred_element_type=jnp.float32)
        adv_ref[...] = jnp.dot(h, ad_ref[...], preferred_element_type=jnp.float32)

    def run(x, W, att_s, att_d):
        return pl.pallas_call(
            body,
            grid=(GRID,),
            in_specs=[
                pl.BlockSpec((BLK, K), lambda i: (i, 0)),
                pl.BlockSpec((K, Dout), lambda i: (0, 0)),
                pl.BlockSpec((Dout, 1), lambda i: (0, 0)),
                pl.BlockSpec((Dout, 1), lambda i: (0, 0)),
            ],
            out_specs=[
                pl.BlockSpec((BLK, Dout), lambda i: (i, 0)),
                pl.BlockSpec((BLK, 1), lambda i: (i, 0)),
                pl.BlockSpec((BLK, 1), lambda i: (i, 0)),
            ],
            out_shape=[
                jax.ShapeDtypeStruct((N, Dout), jnp.float32),
                jax.ShapeDtypeStruct((N, 1), jnp.float32),
                jax.ShapeDtypeStruct((N, 1), jnp.float32),
            ],
        )(x, W, att_s, att_d)

    return run


_tc_dense_1 = _make_tc_dense(128, 256, False)
_tc_dense_2 = _make_tc_dense(256, 128, True)


def kernel(x, edge_index, W1, att_src1, att_dst1, b1, W2, att_src2, att_dst2, b2):
    ei = edge_index.astype(jnp.int32)
    esrc, edst = ei[0], ei[1]
    src_lists, dloc_lists, counts = _sc_partition(esrc, edst)

    h1, as1, ad1 = _tc_dense_1(x, W1, att_src1.reshape(-1, 1), att_dst1.reshape(-1, 1))
    ad1p = jnp.pad(ad1.reshape(-1), (0, ADP - N))
    out1 = _sc_edge_256(h1, src_lists, dloc_lists, counts,
                        as1.reshape(-1), ad1p, b1)

    h2, as2, ad2 = _tc_dense_2(out1[:N], W2, att_src2.reshape(-1, 1), att_dst2.reshape(-1, 1))
    ad2p = jnp.pad(ad2.reshape(-1), (0, ADP - N))
    out2 = _sc_edge_128(h2, src_lists, dloc_lists, counts,
                        as2.reshape(-1), ad2p, b2)
    return out2[:N]


# speedup vs baseline: 1.0665x; 1.0665x over previous
import jax
import jax.numpy as jnp
from jax import lax
from jax.experimental import pallas as pl
from jax.experimental.pallas import tpu as pltpu
from jax.experimental.pallas import tpu_sc as plsc

N = 10000
E = 320000
NC = 2
NS = 16
NW = NC * NS
L = 16
NEG = 0.2

RANGE = 320
PADR = 336
SENT = RANGE
OUT_P = NW * RANGE
ADP = OUT_P

SLABP = 2000
BUFP = 4224
ECAP = 324096
LSLAB = 2048

_mesh = plsc.VectorSubcoreMesh(core_axis_name="c", subcore_axis_name="s")
_params = pltpu.CompilerParams(needs_layout_passes=False)


def _leaky(v):
    return jnp.where(v >= 0, v, NEG * v)


def _wid():
    return lax.axis_index("s") * NC + lax.axis_index("c")


def _pbody(esrc_hbm, edst_hbm, src_out, dloc_out, cnt_out,
           se0, de0, se1, de1, src_b, dloc_b, cv, sem0, sem1):
    wid = _wid()
    base = wid * RANGE

    def issue(q, se, de, sem):
        off = pl.multiple_of(q * SLABP, 8)
        pltpu.async_copy(esrc_hbm.at[pl.ds(off, SLABP)], se, sem)
        pltpu.async_copy(edst_hbm.at[pl.ds(off, SLABP)], de, sem)

    def wait(se, de, sem):
        pltpu.make_async_copy(esrc_hbm.at[pl.ds(0, SLABP)], se, sem).wait()
        pltpu.make_async_copy(edst_hbm.at[pl.ds(0, SLABP)], de, sem).wait()

    def scan(se, de, cnt):
        def vec(i, cnt):
            s_v = se[pl.ds(i * L, L)]
            d_v = de[pl.ds(i * L, L)]
            dl = d_v - base
            m = (d_v >= base) & (d_v < base + RANGE)
            plsc.store_compressed(src_b.at[pl.ds(cnt, L)], s_v, mask=m)
            plsc.store_compressed(dloc_b.at[pl.ds(cnt, L)], dl, mask=m)
            return cnt + plsc.all_reduce_population_count(m)[0]
        return lax.fori_loop(0, SLABP // L, vec, cnt)

    def flush(carry):
        cnt, flushed = carry
        full = cnt >= 2048

        @pl.when(full)
        def _():
            off = pl.multiple_of(wid * ECAP + flushed, 2048)
            pltpu.sync_copy(src_b.at[pl.ds(0, 2048)],
                            src_out.at[pl.ds(off, 2048)])
            pltpu.sync_copy(dloc_b.at[pl.ds(0, 2048)],
                            dloc_out.at[pl.ds(off, 2048)])
        nmv = jnp.where(full, (cnt - 2048 + (L - 1)) // L, 0)

        def mv(i, _):
            src_b[pl.ds(i * L, L)] = src_b[pl.ds(2048 + i * L, L)]
            dloc_b[pl.ds(i * L, L)] = dloc_b[pl.ds(2048 + i * L, L)]
            return 0
        lax.fori_loop(0, nmv, mv, 0)
        return (jnp.where(full, cnt - 2048, cnt),
                jnp.where(full, flushed + 2048, flushed))

    NPAIR = (E // SLABP) // 2
    issue(0, se0, de0, sem0)

    def pair(p, carry):
        wait(se0, de0, sem0)
        issue(2 * p + 1, se1, de1, sem1)
        carry = flush((scan(se0, de0, carry[0]), carry[1]))
        wait(se1, de1, sem1)

        @pl.when(p + 1 < NPAIR)
        def _():
            issue(2 * p + 2, se0, de0, sem0)
        return flush((scan(se1, de1, carry[0]), carry[1]))
    cnt, flushed = lax.fori_loop(0, NPAIR, pair,
                                 (jnp.int32(0), jnp.int32(0)))

    zi = jnp.zeros((L,), jnp.int32)
    sent = jnp.full((L,), SENT, jnp.int32)
    for t in range(8):
        src_b[pl.ds(cnt + t * L, L)] = zi
        dloc_b[pl.ds(cnt + t * L, L)] = sent
    cnt128 = ((cnt + 127) // 128) * 128
    off = pl.multiple_of(wid * ECAP + flushed, 2048)
    pltpu.sync_copy(src_b, src_out.at[pl.ds(off, BUFP)])
    pltpu.sync_copy(dloc_b, dloc_out.at[pl.ds(off, BUFP)])
    cv[pl.ds(0, L)] = jnp.broadcast_to(flushed + cnt128, (L,))
    pltpu.sync_copy(cv, cnt_out.at[pl.ds(pl.multiple_of(wid * L, L), L)])


_sc_partition = pl.kernel(
    _pbody,
    out_type=(
        jax.ShapeDtypeStruct((NW * ECAP,), jnp.int32),
        jax.ShapeDtypeStruct((NW * ECAP,), jnp.int32),
        jax.ShapeDtypeStruct((NW * L,), jnp.int32),
    ),
    mesh=_mesh,
    compiler_params=_params,
    scratch_types=[
        pltpu.VMEM((SLABP,), jnp.int32),
        pltpu.VMEM((SLABP,), jnp.int32),
        pltpu.VMEM((SLABP,), jnp.int32),
        pltpu.VMEM((SLABP,), jnp.int32),
        pltpu.VMEM((BUFP,), jnp.int32),
        pltpu.VMEM((BUFP,), jnp.int32),
        pltpu.VMEM((L,), jnp.int32),
        pltpu.SemaphoreType.DMA,
        pltpu.SemaphoreType.DMA,
    ],
)


def _make_sc_edge(D, KCH):
    KV = D // L

    def body(h_hbm, slh, dlh, cth, as_hbm, adp_hbm, b_hbm, out_hbm,
             acc, den, inv320, a_s_v, a_d_own, bias_v,
             src_l, dloc_l, w_l, rows0, rows1, si0, di0, wc0, si1, di1, wc1,
             counts_v, stage16, sem0, sem1):
        wid = _wid()
        base = wid * RANGE
        io = lax.iota(jnp.int32, L)
        zf = jnp.zeros((L,), jnp.float32)

        pltpu.sync_copy(as_hbm, a_s_v)
        pltpu.sync_copy(adp_hbm.at[pl.ds(pl.multiple_of(base, RANGE), RANGE)],
                        a_d_own.at[pl.ds(0, RANGE)])
        pltpu.sync_copy(b_hbm, bias_v)
        pltpu.sync_copy(cth, counts_v)
        cnt128 = counts_v[pl.ds(wid * L, L)][0]

        def zrow(j, _):
            for k in range(KV):
                acc[j, pl.ds(k * L, L)] = zf
            return 0
        lax.fori_loop(0, PADR, zrow, 0)

        def zden(t, _):
            den[pl.ds(t * L, L)] = zf
            return 0
        lax.fori_loop(0, 352 // L, zden, 0)

        def mx(i, mv):
            return jnp.maximum(mv, a_s_v[pl.ds(i * L, L)])
        mv = lax.fori_loop(0, N // L, mx, jnp.full((L,), -3.0e38, jnp.float32))
        stage16[pl.ds(0, L)] = mv
        for sh in (8, 4, 2, 1):
            mv = jnp.maximum(mv, plsc.load_gather(stage16, [io ^ sh]))
            stage16[pl.ds(0, L)] = mv
        Msp = mv

        def stage_issue(c, si, di, wc, rows, sem):
            for t in range(KCH // L):
                si[pl.ds(t * L, L)] = src_l[pl.ds(c * KCH + t * L, L)]
                di[pl.ds(t * L, L)] = dloc_l[pl.ds(c * KCH + t * L, L)]
                wc[pl.ds(t * L, L)] = w_l[pl.ds(c * KCH + t * L, L)]
            pltpu.async_copy(h_hbm.at[si], rows, sem)

        def process(di, wc, rows):
            def grp(g, _):
                dl_v = di[pl.ds(g * L, L)]
                w_v = wc[pl.ds(g * L, L)]
                d_all = [dl_v[j] for j in range(L)]
                wsp_all = [jnp.broadcast_to(w_v[j], (L,)) for j in range(L)]
                for j in range(L):
                    d_j = d_all[j]
                    wsp = wsp_all[j]
                    r = g * L + j
                    vals = [rows[r, pl.ds(k * L, L)] * wsp for k in range(KV)]
                    for k in range(KV):
                        plsc.addupdate(acc.at[d_j, pl.ds(k * L, L)], vals[k])
                    dbase = (d_j // L) * L
                    unit = jnp.where(io == (d_j % L), wsp, 0.0)
                    plsc.addupdate(den.at[pl.ds(dbase, L)], unit)
                return 0
            lax.fori_loop(0, KCH // L, grp, 0)

        def do_chunks(nproc):
            def passa(i, _):
                sg = src_l[pl.ds(i * L, L)]
                dl = dloc_l[pl.ds(i * L, L)]
                asg = plsc.load_gather(a_s_v, [sg])
                adg = plsc.load_gather(a_d_own, [dl])
                e = _leaky(asg + adg)
                mt = _leaky(Msp + adg)
                w_l[pl.ds(i * L, L)] = jnp.exp(e - mt)
                return 0
            lax.fori_loop(0, nproc // L, passa, 0)

            npair = nproc // (2 * KCH)
            stage_issue(0, si0, di0, wc0, rows0, sem0)

            def pairb(p, _):
                pltpu.make_async_copy(h_hbm.at[si0], rows0, sem0).wait()
                stage_issue(2 * p + 1, si1, di1, wc1, rows1, sem1)
                process(di0, wc0, rows0)
                pltpu.make_async_copy(h_hbm.at[si1], rows1, sem1).wait()

                @pl.when(p + 1 < npair)
                def _():
                    stage_issue(2 * p + 2, si0, di0, wc0, rows0, sem0)
                process(di1, wc1, rows1)
                return 0
            lax.fori_loop(0, npair, pairb, 0)

        def selfapp(t, cnt):
            g = base + t * L + io
            m = g < N
            plsc.store_compressed(src_l.at[pl.ds(cnt, L)], g, mask=m)
            plsc.store_compressed(dloc_l.at[pl.ds(cnt, L)], t * L + io, mask=m)
            return cnt + plsc.all_reduce_population_count(m)[0]
        cnt = lax.fori_loop(0, RANGE // L, selfapp, jnp.int32(0))
        zi = jnp.zeros((L,), jnp.int32)
        sent = jnp.full((L,), SENT, jnp.int32)
        for t in range(8):
            src_l[pl.ds(cnt + t * L, L)] = zi
            dloc_l[pl.ds(cnt + t * L, L)] = sent
        do_chunks(((cnt + 127) // 128) * 128)

        nslab = (cnt128 + (LSLAB - 1)) // LSLAB

        def slab(q, _):
            off = pl.multiple_of(wid * ECAP + q * LSLAB, LSLAB)
            pltpu.sync_copy(slh.at[pl.ds(off, LSLAB)],
                            src_l.at[pl.ds(0, LSLAB)])
            pltpu.sync_copy(dlh.at[pl.ds(off, LSLAB)],
                            dloc_l.at[pl.ds(0, LSLAB)])
            do_chunks(jnp.minimum(LSLAB, cnt128 - q * LSLAB))
            return 0
        lax.fori_loop(0, nslab, slab, 0)

        for t in range(RANGE // L):
            dv = den[pl.ds(t * L, L)]
            inv320[pl.ds(t * L, L)] = 1.0 / (dv + 1e-16)

        def fin(j, _):
            isp = plsc.load_gather(inv320, [jnp.broadcast_to(j, (L,))])
            vals = [acc[j, pl.ds(k * L, L)] * isp + bias_v[pl.ds(k * L, L)]
                    for k in range(KV)]
            for k in range(KV):
                acc[j, pl.ds(k * L, L)] = vals[k]
            return 0
        lax.fori_loop(0, RANGE, fin, 0)
        for g in range(RANGE // 64):
            pltpu.sync_copy(acc.at[pl.ds(g * 64, 64)],
                            out_hbm.at[pl.ds(pl.multiple_of(base + g * 64, 64), 64)])

    return pl.kernel(
        body,
        out_type=jax.ShapeDtypeStruct((OUT_P, D), jnp.float32),
        mesh=_mesh,
        compiler_params=_params,
        scratch_types=[
            pltpu.VMEM((PADR, D), jnp.float32),
            pltpu.VMEM((352,), jnp.float32),
            pltpu.VMEM((RANGE,), jnp.float32),
            pltpu.VMEM((N,), jnp.float32),
            pltpu.VMEM((PADR,), jnp.float32),
            pltpu.VMEM((D,), jnp.float32),
            pltpu.VMEM((LSLAB + 128,), jnp.int32),
            pltpu.VMEM((LSLAB + 128,), jnp.int32),
            pltpu.VMEM((LSLAB + 128,), jnp.float32),
            pltpu.VMEM((KCH, D), jnp.float32),
            pltpu.VMEM((KCH, D), jnp.float32),
            pltpu.VMEM((KCH,), jnp.int32),
            pltpu.VMEM((KCH,), jnp.int32),
            pltpu.VMEM((KCH,), jnp.float32),
            pltpu.VMEM((KCH,), jnp.int32),
            pltpu.VMEM((KCH,), jnp.int32),
            pltpu.VMEM((KCH,), jnp.float32),
            pltpu.VMEM((NW * L,), jnp.int32),
            pltpu.VMEM((L,), jnp.float32),
            pltpu.SemaphoreType.DMA,
            pltpu.SemaphoreType.DMA,
        ],
    )


_sc_edge_256 = _make_sc_edge(256, 32)
_sc_edge_128 = _make_sc_edge(128, 64)


def _make_tc_dense(K, Dout, relu_in):
    BLK = 400
    GRID = N // BLK

    def body(x_ref, w_ref, as_ref, ad_ref, h_ref, asv_ref, adv_ref):
        xb = x_ref[...]
        if relu_in:
            xb = jnp.maximum(xb, 0.0)
        h = jnp.dot(xb, w_ref[...], preferred_element_type=jnp.float32)
        h_ref[...] = h
        asv_ref[...] = jnp.dot(h, as_ref[...], preferred_element_type=jnp.float32)
        adv_ref[...] = jnp.dot(h, ad_ref[...], preferred_element_type=jnp.float32)

    def run(x, W, att_s, att_d):
        return pl.pallas_call(
            body,
            grid=(GRID,),
            in_specs=[
                pl.BlockSpec((BLK, K), lambda i: (i, 0)),
                pl.BlockSpec((K, Dout), lambda i: (0, 0)),
                pl.BlockSpec((Dout, 1), lambda i: (0, 0)),
                pl.BlockSpec((Dout, 1), lambda i: (0, 0)),
            ],
            out_specs=[
                pl.BlockSpec((BLK, Dout), lambda i: (i, 0)),
                pl.BlockSpec((BLK, 1), lambda i: (i, 0)),
                pl.BlockSpec((BLK, 1), lambda i: (i, 0)),
            ],
            out_shape=[
                jax.ShapeDtypeStruct((N, Dout), jnp.float32),
                jax.ShapeDtypeStruct((N, 1), jnp.float32),
                jax.ShapeDtypeStruct((N, 1), jnp.float32),
            ],
        )(x, W, att_s, att_d)

    return run


_tc_dense_1 = _make_tc_dense(128, 256, False)
_tc_dense_2 = _make_tc_dense(256, 128, True)


def kernel(x, edge_index, W1, att_src1, att_dst1, b1, W2, att_src2, att_dst2, b2):
    ei = edge_index.astype(jnp.int32)
    esrc, edst = ei[0], ei[1]
    src_lists, dloc_lists, counts = _sc_partition(esrc, edst)

    h1, as1, ad1 = _tc_dense_1(x, W1, att_src1.reshape(-1, 1), att_dst1.reshape(-1, 1))
    ad1p = jnp.pad(ad1.reshape(-1), (0, ADP - N))
    out1 = _sc_edge_256(h1, src_lists, dloc_lists, counts,
                        as1.reshape(-1), ad1p, b1)

    h2, as2, ad2 = _tc_dense_2(out1[:N], W2, att_src2.reshape(-1, 1), att_dst2.reshape(-1, 1))
    ad2p = jnp.pad(ad2.reshape(-1), (0, ADP - N))
    out2 = _sc_edge_128(h2, src_lists, dloc_lists, counts,
                        as2.reshape(-1), ad2p, b2)
    return out2[:N]


# speedup vs baseline: 1.0724x; 1.0055x over previous
import jax
import jax.numpy as jnp
from jax import lax
from jax.experimental import pallas as pl
from jax.experimental.pallas import tpu as pltpu
from jax.experimental.pallas import tpu_sc as plsc

N = 10000
E = 320000
NC = 2
NS = 16
NW = NC * NS
L = 16
NEG = 0.2

RANGE = 320
PADR = 336
SENT = RANGE
OUT_P = NW * RANGE
ADP = OUT_P

SLABP = 2000
BUFP = 4224
ECAP = 324096
LSLAB = 4096

_mesh = plsc.VectorSubcoreMesh(core_axis_name="c", subcore_axis_name="s")
_params = pltpu.CompilerParams(needs_layout_passes=False)


def _leaky(v):
    return jnp.where(v >= 0, v, NEG * v)


def _wid():
    return lax.axis_index("s") * NC + lax.axis_index("c")


def _pbody(esrc_hbm, edst_hbm, src_out, dloc_out, cnt_out,
           se0, de0, se1, de1, src_b, dloc_b, cv, sem0, sem1):
    wid = _wid()
    base = wid * RANGE

    def issue(q, se, de, sem):
        off = pl.multiple_of(q * SLABP, 8)
        pltpu.async_copy(esrc_hbm.at[pl.ds(off, SLABP)], se, sem)
        pltpu.async_copy(edst_hbm.at[pl.ds(off, SLABP)], de, sem)

    def wait(se, de, sem):
        pltpu.make_async_copy(esrc_hbm.at[pl.ds(0, SLABP)], se, sem).wait()
        pltpu.make_async_copy(edst_hbm.at[pl.ds(0, SLABP)], de, sem).wait()

    def scan(se, de, cnt):
        def vec(i, cnt):
            s_v = se[pl.ds(i * L, L)]
            d_v = de[pl.ds(i * L, L)]
            dl = d_v - base
            m = (d_v >= base) & (d_v < base + RANGE)
            plsc.store_compressed(src_b.at[pl.ds(cnt, L)], s_v, mask=m)
            plsc.store_compressed(dloc_b.at[pl.ds(cnt, L)], dl, mask=m)
            return cnt + plsc.all_reduce_population_count(m)[0]
        return lax.fori_loop(0, SLABP // L, vec, cnt)

    def flush(carry):
        cnt, flushed = carry
        full = cnt >= 2048

        @pl.when(full)
        def _():
            off = pl.multiple_of(wid * ECAP + flushed, 2048)
            pltpu.sync_copy(src_b.at[pl.ds(0, 2048)],
                            src_out.at[pl.ds(off, 2048)])
            pltpu.sync_copy(dloc_b.at[pl.ds(0, 2048)],
                            dloc_out.at[pl.ds(off, 2048)])
        nmv = jnp.where(full, (cnt - 2048 + (L - 1)) // L, 0)

        def mv(i, _):
            src_b[pl.ds(i * L, L)] = src_b[pl.ds(2048 + i * L, L)]
            dloc_b[pl.ds(i * L, L)] = dloc_b[pl.ds(2048 + i * L, L)]
            return 0
        lax.fori_loop(0, nmv, mv, 0)
        return (jnp.where(full, cnt - 2048, cnt),
                jnp.where(full, flushed + 2048, flushed))

    NPAIR = (E // SLABP) // 2
    issue(0, se0, de0, sem0)

    def pair(p, carry):
        wait(se0, de0, sem0)
        issue(2 * p + 1, se1, de1, sem1)
        carry = flush((scan(se0, de0, carry[0]), carry[1]))
        wait(se1, de1, sem1)

        @pl.when(p + 1 < NPAIR)
        def _():
            issue(2 * p + 2, se0, de0, sem0)
        return flush((scan(se1, de1, carry[0]), carry[1]))
    cnt, flushed = lax.fori_loop(0, NPAIR, pair,
                                 (jnp.int32(0), jnp.int32(0)))

    zi = jnp.zeros((L,), jnp.int32)
    sent = jnp.full((L,), SENT, jnp.int32)
    for t in range(8):
        src_b[pl.ds(cnt + t * L, L)] = zi
        dloc_b[pl.ds(cnt + t * L, L)] = sent
    cnt128 = ((cnt + 127) // 128) * 128
    off = pl.multiple_of(wid * ECAP + flushed, 2048)
    pltpu.sync_copy(src_b, src_out.at[pl.ds(off, BUFP)])
    pltpu.sync_copy(dloc_b, dloc_out.at[pl.ds(off, BUFP)])
    cv[pl.ds(0, L)] = jnp.broadcast_to(flushed + cnt128, (L,))
    pltpu.sync_copy(cv, cnt_out.at[pl.ds(pl.multiple_of(wid * L, L), L)])


_sc_partition = pl.kernel(
    _pbody,
    out_type=(
        jax.ShapeDtypeStruct((NW * ECAP,), jnp.int32),
        jax.ShapeDtypeStruct((NW * ECAP,), jnp.int32),
        jax.ShapeDtypeStruct((NW * L,), jnp.int32),
    ),
    mesh=_mesh,
    compiler_params=_params,
    scratch_types=[
        pltpu.VMEM((SLABP,), jnp.int32),
        pltpu.VMEM((SLABP,), jnp.int32),
        pltpu.VMEM((SLABP,), jnp.int32),
        pltpu.VMEM((SLABP,), jnp.int32),
        pltpu.VMEM((BUFP,), jnp.int32),
        pltpu.VMEM((BUFP,), jnp.int32),
        pltpu.VMEM((L,), jnp.int32),
        pltpu.SemaphoreType.DMA,
        pltpu.SemaphoreType.DMA,
    ],
)


def _make_sc_edge(D, KCH):
    KV = D // L

    def body(h_hbm, slh, dlh, cth, as_hbm, adp_hbm, b_hbm, out_hbm,
             acc, den, inv320, a_s_v, a_d_own, bias_v,
             src_l, dloc_l, w_l, rows0, rows1, si0, di0, wc0, si1, di1, wc1,
             counts_v, stage16, sem0, sem1):
        wid = _wid()
        base = wid * RANGE
        io = lax.iota(jnp.int32, L)
        zf = jnp.zeros((L,), jnp.float32)

        pltpu.sync_copy(as_hbm, a_s_v)
        pltpu.sync_copy(adp_hbm.at[pl.ds(pl.multiple_of(base, RANGE), RANGE)],
                        a_d_own.at[pl.ds(0, RANGE)])
        pltpu.sync_copy(b_hbm, bias_v)
        pltpu.sync_copy(cth, counts_v)
        cnt128 = counts_v[pl.ds(wid * L, L)][0]

        def zrow(j, _):
            for k in range(KV):
                acc[j, pl.ds(k * L, L)] = zf
            return 0
        lax.fori_loop(0, PADR, zrow, 0)

        def zden(t, _):
            den[pl.ds(t * L, L)] = zf
            return 0
        lax.fori_loop(0, 352 // L, zden, 0)

        def mx(i, mv):
            return jnp.maximum(mv, a_s_v[pl.ds(i * L, L)])
        mv = lax.fori_loop(0, N // L, mx, jnp.full((L,), -3.0e38, jnp.float32))
        stage16[pl.ds(0, L)] = mv
        for sh in (8, 4, 2, 1):
            mv = jnp.maximum(mv, plsc.load_gather(stage16, [io ^ sh]))
            stage16[pl.ds(0, L)] = mv
        Msp = mv

        def stage_issue(c, si, di, wc, rows, sem):
            for t in range(KCH // L):
                si[pl.ds(t * L, L)] = src_l[pl.ds(c * KCH + t * L, L)]
                di[pl.ds(t * L, L)] = dloc_l[pl.ds(c * KCH + t * L, L)]
                wc[pl.ds(t * L, L)] = w_l[pl.ds(c * KCH + t * L, L)]
            pltpu.async_copy(h_hbm.at[si], rows, sem)

        def process(di, wc, rows):
            def grp(g, _):
                dl_v = di[pl.ds(g * L, L)]
                w_v = wc[pl.ds(g * L, L)]
                d_all = [dl_v[j] for j in range(L)]
                wsp_all = [jnp.broadcast_to(w_v[j], (L,)) for j in range(L)]
                for j in range(L):
                    d_j = d_all[j]
                    wsp = wsp_all[j]
                    r = g * L + j
                    vals = [rows[r, pl.ds(k * L, L)] * wsp for k in range(KV)]
                    for k in range(KV):
                        plsc.addupdate(acc.at[d_j, pl.ds(k * L, L)], vals[k])
                    dbase = (d_j // L) * L
                    unit = jnp.where(io == (d_j % L), wsp, 0.0)
                    plsc.addupdate(den.at[pl.ds(dbase, L)], unit)
                return 0
            lax.fori_loop(0, KCH // L, grp, 0)

        def do_chunks(nproc):
            def passa(i, _):
                sg = src_l[pl.ds(i * L, L)]
                dl = dloc_l[pl.ds(i * L, L)]
                asg = plsc.load_gather(a_s_v, [sg])
                adg = plsc.load_gather(a_d_own, [dl])
                e = _leaky(asg + adg)
                mt = _leaky(Msp + adg)
                w_l[pl.ds(i * L, L)] = jnp.exp(e - mt)
                return 0
            lax.fori_loop(0, nproc // L, passa, 0)

            npair = nproc // (2 * KCH)
            stage_issue(0, si0, di0, wc0, rows0, sem0)

            def pairb(p, _):
                pltpu.make_async_copy(h_hbm.at[si0], rows0, sem0).wait()
                stage_issue(2 * p + 1, si1, di1, wc1, rows1, sem1)
                process(di0, wc0, rows0)
                pltpu.make_async_copy(h_hbm.at[si1], rows1, sem1).wait()

                @pl.when(p + 1 < npair)
                def _():
                    stage_issue(2 * p + 2, si0, di0, wc0, rows0, sem0)
                process(di1, wc1, rows1)
                return 0
            lax.fori_loop(0, npair, pairb, 0)

        def selfapp(t, cnt):
            g = base + t * L + io
            m = g < N
            plsc.store_compressed(src_l.at[pl.ds(cnt, L)], g, mask=m)
            plsc.store_compressed(dloc_l.at[pl.ds(cnt, L)], t * L + io, mask=m)
            return cnt + plsc.all_reduce_population_count(m)[0]
        cnt = lax.fori_loop(0, RANGE // L, selfapp, jnp.int32(0))
        zi = jnp.zeros((L,), jnp.int32)
        sent = jnp.full((L,), SENT, jnp.int32)
        for t in range(8):
            src_l[pl.ds(cnt + t * L, L)] = zi
            dloc_l[pl.ds(cnt + t * L, L)] = sent
        do_chunks(((cnt + 127) // 128) * 128)

        nslab = (cnt128 + (LSLAB - 1)) // LSLAB

        def slab(q, _):
            off = pl.multiple_of(wid * ECAP + q * LSLAB, LSLAB)
            pltpu.sync_copy(slh.at[pl.ds(off, LSLAB)],
                            src_l.at[pl.ds(0, LSLAB)])
            pltpu.sync_copy(dlh.at[pl.ds(off, LSLAB)],
                            dloc_l.at[pl.ds(0, LSLAB)])
            do_chunks(jnp.minimum(LSLAB, cnt128 - q * LSLAB))
            return 0
        lax.fori_loop(0, nslab, slab, 0)

        for t in range(RANGE // L):
            dv = den[pl.ds(t * L, L)]
            inv320[pl.ds(t * L, L)] = 1.0 / (dv + 1e-16)

        def fin(j, _):
            isp = plsc.load_gather(inv320, [jnp.broadcast_to(j, (L,))])
            vals = [acc[j, pl.ds(k * L, L)] * isp + bias_v[pl.ds(k * L, L)]
                    for k in range(KV)]
            for k in range(KV):
                acc[j, pl.ds(k * L, L)] = vals[k]
            return 0
        lax.fori_loop(0, RANGE, fin, 0)
        for g in range(RANGE // 64):
            pltpu.sync_copy(acc.at[pl.ds(g * 64, 64)],
                            out_hbm.at[pl.ds(pl.multiple_of(base + g * 64, 64), 64)])

    return pl.kernel(
        body,
        out_type=jax.ShapeDtypeStruct((OUT_P, D), jnp.float32),
        mesh=_mesh,
        compiler_params=_params,
        scratch_types=[
            pltpu.VMEM((PADR, D), jnp.float32),
            pltpu.VMEM((352,), jnp.float32),
            pltpu.VMEM((RANGE,), jnp.float32),
            pltpu.VMEM((N,), jnp.float32),
            pltpu.VMEM((PADR,), jnp.float32),
            pltpu.VMEM((D,), jnp.float32),
            pltpu.VMEM((LSLAB + 128,), jnp.int32),
            pltpu.VMEM((LSLAB + 128,), jnp.int32),
            pltpu.VMEM((LSLAB + 128,), jnp.float32),
            pltpu.VMEM((KCH, D), jnp.float32),
            pltpu.VMEM((KCH, D), jnp.float32),
            pltpu.VMEM((KCH,), jnp.int32),
            pltpu.VMEM((KCH,), jnp.int32),
            pltpu.VMEM((KCH,), jnp.float32),
            pltpu.VMEM((KCH,), jnp.int32),
            pltpu.VMEM((KCH,), jnp.int32),
            pltpu.VMEM((KCH,), jnp.float32),
            pltpu.VMEM((NW * L,), jnp.int32),
            pltpu.VMEM((L,), jnp.float32),
            pltpu.SemaphoreType.DMA,
            pltpu.SemaphoreType.DMA,
        ],
    )


_sc_edge_256 = _make_sc_edge(256, 32)
_sc_edge_128 = _make_sc_edge(128, 64)


def _make_tc_dense(K, Dout, relu_in):
    BLK = 400
    GRID = N // BLK

    def body(x_ref, w_ref, as_ref, ad_ref, h_ref, asv_ref, adv_ref):
        xb = x_ref[...]
        if relu_in:
            xb = jnp.maximum(xb, 0.0)
        h = jnp.dot(xb, w_ref[...], preferred_element_type=jnp.float32)
        h_ref[...] = h
        asv_ref[...] = jnp.dot(h, as_ref[...], preferred_element_type=jnp.float32)
        adv_ref[...] = jnp.dot(h, ad_ref[...], preferred_element_type=jnp.float32)

    def run(x, W, att_s, att_d):
        return pl.pallas_call(
            body,
            grid=(GRID,),
            in_specs=[
                pl.BlockSpec((BLK, K), lambda i: (i, 0)),
                pl.BlockSpec((K, Dout), lambda i: (0, 0)),
                pl.BlockSpec((Dout, 1), lambda i: (0, 0)),
                pl.BlockSpec((Dout, 1), lambda i: (0, 0)),
            ],
            out_specs=[
                pl.BlockSpec((BLK, Dout), lambda i: (i, 0)),
                pl.BlockSpec((BLK, 1), lambda i: (i, 0)),
                pl.BlockSpec((BLK, 1), lambda i: (i, 0)),
            ],
            out_shape=[
                jax.ShapeDtypeStruct((N, Dout), jnp.float32),
                jax.ShapeDtypeStruct((N, 1), jnp.float32),
                jax.ShapeDtypeStruct((N, 1), jnp.float32),
            ],
        )(x, W, att_s, att_d)

    return run


_tc_dense_1 = _make_tc_dense(128, 256, False)
_tc_dense_2 = _make_tc_dense(256, 128, True)


def kernel(x, edge_index, W1, att_src1, att_dst1, b1, W2, att_src2, att_dst2, b2):
    ei = edge_index.astype(jnp.int32)
    esrc, edst = ei[0], ei[1]
    src_lists, dloc_lists, counts = _sc_partition(esrc, edst)

    h1, as1, ad1 = _tc_dense_1(x, W1, att_src1.reshape(-1, 1), att_dst1.reshape(-1, 1))
    ad1p = jnp.pad(ad1.reshape(-1), (0, ADP - N))
    out1 = _sc_edge_256(h1, src_lists, dloc_lists, counts,
                        as1.reshape(-1), ad1p, b1)

    h2, as2, ad2 = _tc_dense_2(out1[:N], W2, att_src2.reshape(-1, 1), att_dst2.reshape(-1, 1))
    ad2p = jnp.pad(ad2.reshape(-1), (0, ADP - N))
    out2 = _sc_edge_128(h2, src_lists, dloc_lists, counts,
                        as2.reshape(-1), ad2p, b2)
    return out2[:N]
